# baseline (speedup 1.0000x reference)
import jax
import jax.numpy as jnp
from jax.experimental import pallas as pl

BLK = 1024


def _body(gu_ref, gi_ref, xui_ref, guo_ref, gio_ref):
    u = gu_ref[...]
    v = gi_ref[...]
    guo_ref[...] = u
    gio_ref[...] = v
    p = u * v
    s0 = jnp.sum(p[:, :64], axis=1)
    s1 = jnp.sum(p[:, 64:], axis=1)
    xui_ref[...] = jnp.stack([s0, s1], axis=1)


def kernel(gu, gi):
    B, K = gu.shape
    R = B // 2
    gu2 = gu.reshape(R, 2 * K)
    gi2 = gi.reshape(R, 2 * K)
    grid = (R // BLK,)
    xui2, guo, gio = pl.pallas_call(
        _body,
        grid=grid,
        in_specs=[
            pl.BlockSpec((BLK, 2 * K), lambda i: (i, 0)),
            pl.BlockSpec((BLK, 2 * K), lambda i: (i, 0)),
        ],
        out_specs=[
            pl.BlockSpec((BLK, 2), lambda i: (i, 0)),
            pl.BlockSpec((BLK, 2 * K), lambda i: (i, 0)),
            pl.BlockSpec((BLK, 2 * K), lambda i: (i, 0)),
        ],
        out_shape=[
            jax.ShapeDtypeStruct((R, 2), gu.dtype),
            jax.ShapeDtypeStruct((R, 2 * K), gu.dtype),
            jax.ShapeDtypeStruct((R, 2 * K), gi.dtype),
        ],
    )(gu2, gi2)
    return (xui2.reshape(B), guo.reshape(B, K), gio.reshape(B, K))


# speedup vs baseline: 1.6118x; 1.6118x over previous
import jax
import jax.numpy as jnp
from jax.experimental import pallas as pl

BLK = 2048


def _body(gu_ref, gi_ref, xui_ref, guo_ref, gio_ref):
    u = gu_ref[...]
    v = gi_ref[...]
    guo_ref[...] = u
    gio_ref[...] = v
    xui_ref[...] = jnp.sum(u * v, axis=1)


def kernel(gu, gi):
    B, K = gu.shape
    grid = (B // BLK,)
    xui, guo, gio = pl.pallas_call(
        _body,
        grid=grid,
        in_specs=[
            pl.BlockSpec((BLK, K), lambda i: (i, 0)),
            pl.BlockSpec((BLK, K), lambda i: (i, 0)),
        ],
        out_specs=[
            pl.BlockSpec((BLK,), lambda i: (i,)),
            pl.BlockSpec((BLK, K), lambda i: (i, 0)),
            pl.BlockSpec((BLK, K), lambda i: (i, 0)),
        ],
        out_shape=[
            jax.ShapeDtypeStruct((B,), gu.dtype),
            jax.ShapeDtypeStruct((B, K), gu.dtype),
            jax.ShapeDtypeStruct((B, K), gi.dtype),
        ],
    )(gu, gi)
    return (xui, guo, gio)
